# baseline (speedup 1.0000x reference)
import functools

import jax
import jax.numpy as jnp
from jax import lax
from jax.experimental import pallas as pl
from jax.experimental.pallas import tpu as pltpu
from jax.experimental.pallas import tpu_sc as plsc

B = 16384
L = 200
VOCAB = 769
NC = 2
NS = 16
NW = NC * NS
BPW = B // NW
CC = 256
NCH = BPW // CC
GRP = 16
NGROUPS = CC // GRP
NACC = 8


def _pst_kernel(xt_hbm, tab_hbm, out_hbm, tab_v, xb_v, out_v, sem0, sem1):
    wid = lax.axis_index("s") * NC + lax.axis_index("c")
    col0 = wid * BPW
    pltpu.sync_copy(tab_hbm, tab_v)

    sems = (sem0, sem1)
    copies = [None, None]
    for c in range(min(2, NCH)):
        copies[c] = pltpu.async_copy(
            xt_hbm.at[:, pl.ds(col0 + c * CC, CC)], xb_v.at[c], sems[c]
        )

    zero = jnp.zeros((GRP,), jnp.float32)

    for c in range(NCH):
        b = c % 2
        copies[b].wait()

        def group_body(g16, _, b=b, c=c):
            cbase = g16 * GRP

            @plsc.parallel_loop(0, L, NACC, unroll=1, carry=(zero,) * NACC)
            def l_loop(l, accs, b=b, cbase=cbase):
                new = []
                for k in range(NACC):
                    idx = xb_v[b, l + k, pl.ds(cbase, GRP)]
                    new.append(accs[k] + plsc.load_gather(tab_v, [idx]))
                return tuple(new)

            accs = list(l_loop)
            while len(accs) > 1:
                accs = [a + b2 for a, b2 in zip(accs[::2], accs[1::2])]
            out_v[pl.ds(c * CC + cbase, GRP)] = accs[0]
            return _

        lax.fori_loop(0, NGROUPS, group_body, None)

        if c + 2 < NCH:
            copies[b] = pltpu.async_copy(
                xt_hbm.at[:, pl.ds(col0 + (c + 2) * CC, CC)], xb_v.at[b], sems[b]
            )

    pltpu.sync_copy(out_v, out_hbm.at[pl.ds(col0, BPW)])


@jax.jit
def _pst_sum(xt, tab_flat):
    mesh = plsc.VectorSubcoreMesh(core_axis_name="c", subcore_axis_name="s")
    f = pl.kernel(
        _pst_kernel,
        out_type=jax.ShapeDtypeStruct((B,), jnp.float32),
        mesh=mesh,
        scratch_types=[
            pltpu.VMEM((VOCAB,), jnp.float32),
            pltpu.VMEM((2, L, CC), jnp.int32),
            pltpu.VMEM((BPW,), jnp.float32),
            pltpu.SemaphoreType.DMA,
            pltpu.SemaphoreType.DMA,
        ],
        compiler_params=pltpu.CompilerParams(needs_layout_passes=False),
    )
    return f(xt, tab_flat)


def kernel(x, pst_weight, emb_weight):
    xt = x.astype(jnp.int32).T
    return _pst_sum(xt, pst_weight.reshape(-1))


# speedup vs baseline: 1.0823x; 1.0823x over previous
import functools

import jax
import jax.numpy as jnp
from jax import lax
from jax.experimental import pallas as pl
from jax.experimental.pallas import tpu as pltpu
from jax.experimental.pallas import tpu_sc as plsc

B = 16384
L = 200
VOCAB = 769
NC = 2
NS = 16
NW = NC * NS
BPW = B // NW
CC = 128
NCH = BPW // CC
GRP = 16
NGROUPS = CC // GRP
NACC = 8


def _pst_kernel(xt_hbm, tab_hbm, out_hbm, tab_v, xb_v, out_v, sem0, sem1):
    wid = lax.axis_index("s") * NC + lax.axis_index("c")
    col0 = wid * BPW
    pltpu.sync_copy(tab_hbm, tab_v)

    sems = (sem0, sem1)
    copies = [None, None]
    for c in range(min(2, NCH)):
        copies[c] = pltpu.async_copy(
            xt_hbm.at[:, pl.ds(col0 + c * CC, CC)], xb_v.at[c], sems[c]
        )

    zero = jnp.zeros((GRP,), jnp.float32)

    def chunk_compute(c_dyn, b):
        def group_body(g16, _, b=b):
            cbase = g16 * GRP

            @plsc.parallel_loop(0, L, NACC, unroll=2, carry=(zero,) * NACC)
            def l_loop(l, accs, b=b, cbase=cbase):
                new = []
                for k in range(NACC):
                    idx = xb_v[b, l + k, pl.ds(cbase, GRP)]
                    new.append(accs[k] + plsc.load_gather(tab_v, [idx]))
                return tuple(new)

            accs = list(l_loop)
            while len(accs) > 1:
                accs = [a + b2 for a, b2 in zip(accs[::2], accs[1::2])]
            out_v[pl.ds(c_dyn * CC + cbase, GRP)] = accs[0]
            return _

        lax.fori_loop(0, NGROUPS, group_body, None)

    def pair_body(p, _):
        for b in range(2):
            c_dyn = 2 * p + b
            pltpu.make_async_copy(
                xt_hbm.at[:, pl.ds(col0 + c_dyn * CC, CC)], xb_v.at[b], sems[b]
            ).wait()
            chunk_compute(c_dyn, b)

            @pl.when(c_dyn + 2 < NCH)
            def _fire(b=b, c_dyn=c_dyn):
                pltpu.async_copy(
                    xt_hbm.at[:, pl.ds(col0 + (c_dyn + 2) * CC, CC)],
                    xb_v.at[b],
                    sems[b],
                )

        return _

    lax.fori_loop(0, NCH // 2, pair_body, None)

    pltpu.sync_copy(out_v, out_hbm.at[pl.ds(col0, BPW)])


@jax.jit
def _pst_sum(xt, tab_flat):
    mesh = plsc.VectorSubcoreMesh(core_axis_name="c", subcore_axis_name="s")
    f = pl.kernel(
        _pst_kernel,
        out_type=jax.ShapeDtypeStruct((B,), jnp.float32),
        mesh=mesh,
        scratch_types=[
            pltpu.VMEM((VOCAB,), jnp.float32),
            pltpu.VMEM((2, L, CC), jnp.int32),
            pltpu.VMEM((BPW,), jnp.float32),
            pltpu.SemaphoreType.DMA,
            pltpu.SemaphoreType.DMA,
        ],
        compiler_params=pltpu.CompilerParams(needs_layout_passes=False),
    )
    return f(xt, tab_flat)


def kernel(x, pst_weight, emb_weight):
    xt = x.astype(jnp.int32).T
    return _pst_sum(xt, pst_weight.reshape(-1))


# speedup vs baseline: 1.1126x; 1.0280x over previous
import functools

import jax
import jax.numpy as jnp
from jax import lax
from jax.experimental import pallas as pl
from jax.experimental.pallas import tpu as pltpu
from jax.experimental.pallas import tpu_sc as plsc

B = 16384
L = 200
VOCAB = 769
NC = 2
NS = 16
NW = NC * NS
BPW = B // NW
CC = 128
NCH = BPW // CC
GRP = 16
NGROUPS = CC // GRP
NACC = 8


def _pst_kernel(xt_hbm, tab_hbm, out_hbm, tab_v, xb_v, out_v, sem0, sem1):
    wid = lax.axis_index("s") * NC + lax.axis_index("c")
    col0 = wid * BPW

    sems = (sem0, sem1)
    for c in range(min(2, NCH)):
        pltpu.async_copy(
            xt_hbm.at[:, pl.ds(col0 + c * CC, CC)], xb_v.at[c], sems[c]
        )
    pltpu.sync_copy(tab_hbm, tab_v)

    zero = jnp.zeros((GRP,), jnp.float32)

    def chunk_compute(c_dyn, b):
        def group_body(g16, _, b=b):
            cbase = g16 * GRP

            @plsc.parallel_loop(0, L, NACC, unroll=2, carry=(zero,) * NACC)
            def l_loop(l, accs, b=b, cbase=cbase):
                new = []
                for k in range(NACC):
                    idx = xb_v[b, l + k, pl.ds(cbase, GRP)]
                    new.append(accs[k] + plsc.load_gather(tab_v, [idx]))
                return tuple(new)

            accs = list(l_loop)
            while len(accs) > 1:
                accs = [a + b2 for a, b2 in zip(accs[::2], accs[1::2])]
            out_v[pl.ds(c_dyn * CC + cbase, GRP)] = accs[0]
            return _

        lax.fori_loop(0, NGROUPS, group_body, None)

    def pair_body(p, _):
        for b in range(2):
            c_dyn = 2 * p + b
            pltpu.make_async_copy(
                xt_hbm.at[:, pl.ds(col0 + c_dyn * CC, CC)], xb_v.at[b], sems[b]
            ).wait()
            chunk_compute(c_dyn, b)

            @pl.when(c_dyn + 2 < NCH)
            def _fire(b=b, c_dyn=c_dyn):
                pltpu.async_copy(
                    xt_hbm.at[:, pl.ds(col0 + (c_dyn + 2) * CC, CC)],
                    xb_v.at[b],
                    sems[b],
                )

        return _

    lax.fori_loop(0, NCH // 2, pair_body, None)

    pltpu.sync_copy(out_v, out_hbm.at[pl.ds(col0, BPW)])


@jax.jit
def _pst_sum(xt, tab_flat):
    mesh = plsc.VectorSubcoreMesh(core_axis_name="c", subcore_axis_name="s")
    f = pl.kernel(
        _pst_kernel,
        out_type=jax.ShapeDtypeStruct((B,), jnp.float32),
        mesh=mesh,
        scratch_types=[
            pltpu.VMEM((VOCAB,), jnp.float32),
            pltpu.VMEM((2, L, CC), jnp.int32),
            pltpu.VMEM((BPW,), jnp.float32),
            pltpu.SemaphoreType.DMA,
            pltpu.SemaphoreType.DMA,
        ],
        compiler_params=pltpu.CompilerParams(needs_layout_passes=False),
    )
    return f(xt, tab_flat)


def kernel(x, pst_weight, emb_weight):
    xt = x.astype(jnp.int32).T
    return _pst_sum(xt, pst_weight.reshape(-1))
